# Initial kernel scaffold; baseline (speedup 1.0000x reference)
import jax
import jax.numpy as jnp
from jax.experimental import pallas as pl


def kernel(z, codebook, params):
    raise NotImplementedError("write your pallas kernel here")


# speedup vs baseline: 1.6255x; 1.6255x over previous
import jax
import jax.numpy as jnp
from jax.experimental import pallas as pl
from jax.experimental.pallas import tpu as pltpu

_B = 128
_NT = 512
_CD = 512
_L = 16
_ROWS = _B * _L
_EPS = 1e-5

_HIGH = jax.lax.Precision.HIGHEST
_NT_DIMS = (((1,), (1,)), ((), ()))


def _vq_body(flat_ref, cb_ref, fn_ref, cn_ref, idx_ref, quant_ref, loss_ref):
    flat = flat_ref[...]
    cb = cb_ref[...]
    s = jax.lax.dot_general(flat, cb, _NT_DIMS)
    dist = fn_ref[...] + cn_ref[...] - 2.0 * s
    minval = jnp.min(dist, axis=1, keepdims=True)
    jj = jax.lax.broadcasted_iota(jnp.int32, dist.shape, 1)
    idx = jnp.min(jnp.where(dist == minval, jj, _NT), axis=1)
    onehot = (jj == idx[:, None]).astype(jnp.float32)
    quant = jax.lax.dot_general(onehot, cb, (((1,), (0,)), ((), ())),
                                precision=_HIGH)
    diff = quant - flat
    loss_ref[...] = (1.25 * jnp.mean(diff * diff)).reshape(1, 1)
    idx_ref[...] = idx.reshape(_B, _L)
    quant_ref[...] = flat + diff


def _bn_relu(h, g, be):
    m = jnp.mean(h, axis=0, keepdims=True)
    d = h - m
    v = jnp.mean(d * d, axis=0, keepdims=True)
    return jnp.maximum(g[None, :] * d / jnp.sqrt(v + _EPS) + be[None, :], 0.0)


def _fold_body(x_ref, w1_ref, b1_ref, g1_ref, be1_ref,
               w2_ref, b2_ref, g2_ref, be2_ref,
               w3_ref, b3_ref, y_ref):
    x = x_ref[...]
    h = jax.lax.dot_general(x, w1_ref[...], _NT_DIMS)
    h = _bn_relu(h + b1_ref[...][None, :], g1_ref[...], be1_ref[...])
    h = jax.lax.dot_general(h, w2_ref[...], _NT_DIMS)
    h = _bn_relu(h + b2_ref[...][None, :], g2_ref[...], be2_ref[...])
    h = jax.lax.dot_general(h, w3_ref[...], _NT_DIMS)
    y_ref[...] = x + (h + b3_ref[...][None, :])


def _end_body(x_ref, w1_ref, b1_ref, g1_ref, be1_ref, w2_ref, b2_ref,
              out_ref):
    x = x_ref[...]
    mx = jnp.max(x.reshape(_B, _L, _CD), axis=1)
    h = jnp.maximum(mx, 0.0)
    h = jax.lax.dot_general(h, w1_ref[...], _NT_DIMS)
    h = _bn_relu(h + b1_ref[...][None, :], g1_ref[...], be1_ref[...])
    out = jax.lax.dot_general(h, w2_ref[...], _NT_DIMS)
    out_ref[...] = out + b2_ref[...][None, :]


_CP = pltpu.CompilerParams(vmem_limit_bytes=192 * 1024 * 1024)


def kernel(z, codebook, params):
    flat4 = jnp.transpose(z[:, :, :, None], (0, 2, 3, 1)).reshape(-1, _NT)
    fn = jnp.sum(flat4 ** 2, axis=1, keepdims=True)
    cn = jnp.sum(codebook ** 2, axis=1)[None, :]
    idx, quant, loss = pl.pallas_call(
        _vq_body,
        out_shape=(
            jax.ShapeDtypeStruct((_B, _L), jnp.int32),
            jax.ShapeDtypeStruct((_ROWS, _CD), jnp.float32),
            jax.ShapeDtypeStruct((1, 1), jnp.float32),
        ),
        compiler_params=_CP,
    )(flat4, codebook, fn, cn)

    fold = pl.pallas_call(
        _fold_body,
        out_shape=jax.ShapeDtypeStruct((_ROWS, _CD), jnp.float32),
        compiler_params=_CP,
    )
    x = quant
    for name in ('f1', 'f2', 'f3'):
        p = params[name]
        x = fold(x, p['W1'], p['b1'], p['g1'], p['be1'],
                 p['W2'], p['b2'], p['g2'], p['be2'], p['W3'], p['b3'])

    e = params['end']
    out = pl.pallas_call(
        _end_body,
        out_shape=jax.ShapeDtypeStruct((_B, 3 * 2048), jnp.float32),
        compiler_params=_CP,
    )(x, e['W1'], e['b1'], e['g1'], e['be1'], e['W2'], e['b2'])

    return (loss[0, 0], out[:, :, None], idx)
